# baseline (speedup 1.0000x reference)
import functools

import jax
import jax.numpy as jnp
from jax import lax
from jax.experimental import pallas as pl
from jax.experimental.pallas import tpu as pltpu
from jax.experimental.pallas import tpu_sc as plsc

VOCAB = 100000
EMBED = 128
BATCH = 16384

NC = 2
NS = 16
NW = NC * NS
B_PER_W = BATCH // NW
CHUNK = 128
NCHUNK = B_PER_W // CHUNK
GROUPS = CHUNK // 16


def _sc_inner_kernel(a_hbm, b_hbm, sign_hbm, emb_hbm, ctx_hbm, out_hbm,
                     idx_a, idx_b, sg, rows_a, rows_b, stage, lacc,
                     sem_a0, sem_a1, sem_b0, sem_b1):
    wid = lax.axis_index("s") * NC + lax.axis_index("c")
    base = wid * B_PER_W
    iota16 = lax.iota(jnp.int32, 16)
    sems_a = [sem_a0, sem_a1]
    sems_b = [sem_b0, sem_b1]

    def start_chunk(j, buf):
        cb = base + j * CHUNK
        pltpu.sync_copy(a_hbm.at[pl.ds(cb, CHUNK)], idx_a.at[buf])
        pltpu.sync_copy(b_hbm.at[pl.ds(cb, CHUNK)], idx_b.at[buf])
        pltpu.sync_copy(sign_hbm.at[pl.ds(cb, CHUNK)], sg.at[buf])
        cp_a = pltpu.async_copy(emb_hbm.at[idx_a.at[buf]], rows_a.at[buf], sems_a[buf])
        cp_b = pltpu.async_copy(ctx_hbm.at[idx_b.at[buf]], rows_b.at[buf], sems_b[buf])
        return cp_a, cp_b

    def compute_chunk(j, buf):
        ra = rows_a.at[buf]
        rb = rows_b.at[buf]

        def group_body(g, carry2):
            for rr in range(16):
                r = g * 16 + rr
                acc = None
                for c in range(EMBED // 16):
                    va = ra[r, pl.ds(c * 16, 16)]
                    vb = rb[r, pl.ds(c * 16, 16)]
                    p = va * vb
                    acc = p if acc is None else acc + p
                stage[rr, :] = acc
            s = None
            for l in range(16):
                col = plsc.load_gather(stage, [iota16, jnp.full((16,), l, jnp.int32)])
                s = col if s is None else s + col
            z = sg[buf, pl.ds(g * 16, 16)] * s
            w = jnp.exp(-jnp.abs(z))
            x = w / (w + 2.0)
            x2 = x * x
            poly = 1.0 + x2 * (0.33333333 + x2 * (0.2 + x2 * 0.14285714))
            ls = jnp.minimum(z, 0.0) - 2.0 * x * poly
            lacc[:] = lacc[:] + ls
            return carry2

        lax.fori_loop(0, GROUPS, group_body, 0, unroll=False)

    lacc[:] = jnp.zeros((16,), jnp.float32)
    cps = start_chunk(0, 0)
    for j in range(NCHUNK):
        buf = j % 2
        nxt = None
        if j + 1 < NCHUNK:
            nxt = start_chunk(j + 1, (j + 1) % 2)
        cps[0].wait()
        cps[1].wait()
        compute_chunk(j, buf)
        cps = nxt
    pltpu.sync_copy(lacc, out_hbm.at[pl.ds(wid * 16, 16)])


@jax.jit
def _sc_inner(a, b, sign, embeddings, context_embeddings):
    mesh = plsc.VectorSubcoreMesh(core_axis_name="c", subcore_axis_name="s")
    kern = pl.kernel(
        _sc_inner_kernel,
        out_type=jax.ShapeDtypeStruct((NW * 16,), jnp.float32),
        mesh=mesh,
        compiler_params=pltpu.CompilerParams(needs_layout_passes=False),
        scratch_types=[
            pltpu.VMEM((2, CHUNK), jnp.int32),
            pltpu.VMEM((2, CHUNK), jnp.int32),
            pltpu.VMEM((2, CHUNK), jnp.float32),
            pltpu.VMEM((2, CHUNK, EMBED), jnp.float32),
            pltpu.VMEM((2, CHUNK, EMBED), jnp.float32),
            pltpu.VMEM((16, 16), jnp.float32),
            pltpu.VMEM((16,), jnp.float32),
            pltpu.SemaphoreType.DMA,
            pltpu.SemaphoreType.DMA,
            pltpu.SemaphoreType.DMA,
            pltpu.SemaphoreType.DMA,
        ],
    )
    return kern(a, b, sign, embeddings, context_embeddings)


def _loss_body(part_ref, out_ref):
    out_ref[0, 0] = -jnp.sum(part_ref[...]) / BATCH


@jax.jit
def _loss(partials):
    res = pl.pallas_call(
        _loss_body,
        out_shape=jax.ShapeDtypeStruct((1, 1), jnp.float32),
        in_specs=[pl.BlockSpec(memory_space=pltpu.VMEM)],
        out_specs=pl.BlockSpec(memory_space=pltpu.SMEM),
    )(partials.reshape(4, 128))
    return res[0, 0]


def kernel(a, b, sign, embeddings, context_embeddings):
    partials = _sc_inner(a, b, sign, embeddings, context_embeddings)
    return _loss(partials)


# speedup vs baseline: 1.0414x; 1.0414x over previous
import functools

import jax
import jax.numpy as jnp
from jax import lax
from jax.experimental import pallas as pl
from jax.experimental.pallas import tpu as pltpu
from jax.experimental.pallas import tpu_sc as plsc

VOCAB = 100000
EMBED = 128
BATCH = 16384

NC = 2
NS = 16
NW = NC * NS
B_PER_W = BATCH // NW
CHUNK = 128
NCHUNK = B_PER_W // CHUNK
GROUPS = CHUNK // 16


DEPTH = 3


def _sc_inner_kernel(a_hbm, b_hbm, sign_hbm, emb_hbm, ctx_hbm, out_hbm,
                     idx_a, idx_b, sg, rows_a, rows_b, stage, lacc,
                     sem_i0, sem_i1, sem_i2,
                     sem_a0, sem_a1, sem_a2, sem_b0, sem_b1, sem_b2):
    wid = lax.axis_index("s") * NC + lax.axis_index("c")
    base = wid * B_PER_W
    iota16 = lax.iota(jnp.int32, 16)
    sems_a = [sem_a0, sem_a1, sem_a2]
    sems_b = [sem_b0, sem_b1, sem_b2]

    ci = pltpu.async_copy(a_hbm.at[wid], idx_a, sem_i0)
    cj = pltpu.async_copy(b_hbm.at[wid], idx_b, sem_i1)
    ck = pltpu.async_copy(sign_hbm.at[wid], sg, sem_i2)
    ci.wait()
    cj.wait()
    ck.wait()

    def start_chunk(j, buf):
        cp_a = pltpu.async_copy(emb_hbm.at[idx_a.at[j]], rows_a.at[buf], sems_a[buf])
        cp_b = pltpu.async_copy(ctx_hbm.at[idx_b.at[j]], rows_b.at[buf], sems_b[buf])
        return cp_a, cp_b

    def compute_chunk(j, buf):
        ra = rows_a.at[buf]
        rb = rows_b.at[buf]

        def group_body(g, carry2):
            for rr in range(16):
                r = g * 16 + rr
                acc = None
                for c in range(EMBED // 16):
                    va = ra[r, pl.ds(c * 16, 16)]
                    vb = rb[r, pl.ds(c * 16, 16)]
                    p = va * vb
                    acc = p if acc is None else acc + p
                stage[rr, :] = acc
            s = None
            for l in range(16):
                col = plsc.load_gather(stage, [iota16, jnp.full((16,), l, jnp.int32)])
                s = col if s is None else s + col
            z = sg[j, pl.ds(g * 16, 16)] * s
            w = jnp.exp(-jnp.abs(z))
            x = w / (w + 2.0)
            x2 = x * x
            poly = 1.0 + x2 * (0.33333333 + x2 * (0.2 + x2 * 0.14285714))
            ls = jnp.minimum(z, 0.0) - 2.0 * x * poly
            lacc[:] = lacc[:] + ls
            return carry2

        lax.fori_loop(0, GROUPS, group_body, 0, unroll=False)

    lacc[:] = jnp.zeros((16,), jnp.float32)
    cps = [start_chunk(j, j % DEPTH) for j in range(min(DEPTH, NCHUNK))]
    for j in range(NCHUNK):
        buf = j % DEPTH
        cps[j][0].wait()
        cps[j][1].wait()
        compute_chunk(j, buf)
        if j + DEPTH < NCHUNK:
            cps.append(start_chunk(j + DEPTH, buf))
    pltpu.sync_copy(lacc, out_hbm.at[pl.ds(wid * 16, 16)])


@jax.jit
def _sc_inner(a, b, sign, embeddings, context_embeddings):
    mesh = plsc.VectorSubcoreMesh(core_axis_name="c", subcore_axis_name="s")
    kern = pl.kernel(
        _sc_inner_kernel,
        out_type=jax.ShapeDtypeStruct((NW * 16,), jnp.float32),
        mesh=mesh,
        compiler_params=pltpu.CompilerParams(needs_layout_passes=False),
        scratch_types=[
            pltpu.VMEM((NCHUNK, CHUNK), jnp.int32),
            pltpu.VMEM((NCHUNK, CHUNK), jnp.int32),
            pltpu.VMEM((NCHUNK, CHUNK), jnp.float32),
            pltpu.VMEM((DEPTH, CHUNK, EMBED), jnp.float32),
            pltpu.VMEM((DEPTH, CHUNK, EMBED), jnp.float32),
            pltpu.VMEM((16, 16), jnp.float32),
            pltpu.VMEM((16,), jnp.float32),
            pltpu.SemaphoreType.DMA,
            pltpu.SemaphoreType.DMA,
            pltpu.SemaphoreType.DMA,
            pltpu.SemaphoreType.DMA,
            pltpu.SemaphoreType.DMA,
            pltpu.SemaphoreType.DMA,
            pltpu.SemaphoreType.DMA,
            pltpu.SemaphoreType.DMA,
            pltpu.SemaphoreType.DMA,
        ],
    )
    return kern(a.reshape(NW, NCHUNK, CHUNK), b.reshape(NW, NCHUNK, CHUNK),
                sign.reshape(NW, NCHUNK, CHUNK), embeddings, context_embeddings)


def _loss_body(part_ref, out_ref):
    out_ref[0, 0] = -jnp.sum(part_ref[...]) / BATCH


@jax.jit
def _loss(partials):
    res = pl.pallas_call(
        _loss_body,
        out_shape=jax.ShapeDtypeStruct((1, 1), jnp.float32),
        in_specs=[pl.BlockSpec(memory_space=pltpu.VMEM)],
        out_specs=pl.BlockSpec(memory_space=pltpu.SMEM),
    )(partials.reshape(4, 128))
    return res[0, 0]


def kernel(a, b, sign, embeddings, context_embeddings):
    partials = _sc_inner(a, b, sign, embeddings, context_embeddings)
    return _loss(partials)
